# baseline (speedup 1.0000x reference)
import functools

import jax
import jax.numpy as jnp
from jax import lax
from jax.experimental import pallas as pl
from jax.experimental.pallas import tpu as pltpu
from jax.experimental.pallas import tpu_sc as plsc

_RC = 200
_NT = 5000
_NW = 32
_NB = 4
_S = 160


def _sc_copy_item(item_w):
    mesh = plsc.VectorSubcoreMesh(core_axis_name="c", subcore_axis_name="s")

    @functools.partial(
        pl.kernel,
        mesh=mesh,
        out_type=jax.ShapeDtypeStruct(item_w.shape, item_w.dtype),
        scratch_types=(
            [pltpu.VMEM((_RC, 64), jnp.float32)] * _NB
            + [pltpu.SemaphoreType.DMA] * (2 * _NB)
        ),
    )
    def sc_kernel(i_hbm, io_hbm, *scr):
        bufs = scr[:_NB]
        isems = scr[_NB:2 * _NB]
        osems = scr[2 * _NB:3 * _NB]
        wid = lax.axis_index("c") * 16 + lax.axis_index("s")

        def valid(s):
            return (s >= 0) & (s * _NW + wid < _NT)

        def start_in(s, b):
            k = s * _NW + wid

            @pl.when(valid(s))
            def _():
                sl = pl.ds(k * _RC, _RC)
                pltpu.make_async_copy(i_hbm.at[sl], bufs[b], isems[b]).start()

        def finish_start_out(s, b):
            k = s * _NW + wid

            @pl.when(valid(s))
            def _():
                sl = pl.ds(k * _RC, _RC)
                pltpu.make_async_copy(i_hbm.at[sl], bufs[b], isems[b]).wait()
                pltpu.make_async_copy(bufs[b], io_hbm.at[sl], osems[b]).start()

        def wait_out(s, b):
            @pl.when(valid(s))
            def _():
                pltpu.make_async_copy(
                    bufs[b], io_hbm.at[pl.ds(0, _RC)], osems[b]).wait()

        start_in(0, 0)
        start_in(1, 1)

        def body(j, carry):
            for b in range(_NB):
                s = _NB * j + b
                finish_start_out(s, b)
                ns = s + 2
                nb = (b + 2) % _NB
                wait_out(ns - _NB, nb)
                start_in(ns, nb)
            return carry

        lax.fori_loop(0, _S // _NB, body, 0)
        for s in (_S - 2, _S - 1):
            wait_out(s, s % _NB)

    return sc_kernel(item_w)


def _tc_copy_block(src_ref, dst_ref):
    dst_ref[...] = src_ref[...]


def _tc_copy(x, block_rows):
    rows, cols = x.shape
    return pl.pallas_call(
        _tc_copy_block,
        grid=(rows // block_rows,),
        in_specs=[pl.BlockSpec((block_rows, cols), lambda i: (i, 0))],
        out_specs=pl.BlockSpec((block_rows, cols), lambda i: (i, 0)),
        out_shape=jax.ShapeDtypeStruct((rows, cols), x.dtype),
    )(x)


def kernel(user_w, item_w, edge_index):
    del edge_index
    item_out = _sc_copy_item(item_w)
    user_out = _tc_copy(user_w, block_rows=5000)
    return (user_out, item_out)
